# baseline (speedup 1.0000x reference)
import jax
import jax.numpy as jnp
from jax.experimental import pallas as pl


def _stable_zero_div_body(x_ref, y_ref, o_ref):
    x = x_ref[...]
    y = y_ref[...]
    nz = y != 0.0
    inv = jnp.where(nz, 1.0 / jnp.where(nz, y, 1.0), 0.0)
    o_ref[...] = inv * x


def kernel(x, y):
    n = x.shape[0]
    block = 2097152
    out = pl.pallas_call(
        _stable_zero_div_body,
        grid=(n // block,),
        in_specs=[
            pl.BlockSpec((block,), lambda i: (i,)),
            pl.BlockSpec((block,), lambda i: (i,)),
        ],
        out_specs=pl.BlockSpec((block,), lambda i: (i,)),
        out_shape=jax.ShapeDtypeStruct((n,), jnp.float32),
    )(x, y)
    return out
